# Initial kernel scaffold; baseline (speedup 1.0000x reference)
import jax
import jax.numpy as jnp
from jax.experimental import pallas as pl


def kernel(frame, attributes, mask):
    raise NotImplementedError("write your pallas kernel here")


# speedup vs baseline: 4.5349x; 4.5349x over previous
import functools

import jax
import jax.numpy as jnp
from jax import lax
from jax.experimental import pallas as pl
from jax.experimental.pallas import tpu as pltpu
from jax.experimental.pallas import tpu_sc as plsc

B, N, D, K = 4, 2048, 128, 16
L = 16
NC, NS = 2, 16
NW = NC * NS
QPT = (B * N) // NW
TPB = N // QPT
NBLK = N // L
NGRP = QPT // L
IDX_ROWS = (QPT * K) // 128
BIG = 1e30


def _body(cen_hbm, fr9_hbm, attr_hbm, out_c_hbm, out_a_hbm,
          cxv, cyv, czv, penv, frv, idxbuf, coordbuf, rows0, rows1,
          sem0, sem1):
    wid = lax.axis_index("s") * NC + lax.axis_index("c")
    batch = wid // TPB
    q0 = (wid % TPB) * QPT

    pltpu.sync_copy(cen_hbm.at[batch, 0], cxv)
    pltpu.sync_copy(cen_hbm.at[batch, 1], cyv)
    pltpu.sync_copy(cen_hbm.at[batch, 2], czv)
    pltpu.sync_copy(cen_hbm.at[batch, 3], penv)
    pltpu.sync_copy(fr9_hbm.at[batch, :, pl.ds(q0, QPT)], frv)

    lane = lax.iota(jnp.int32, L)

    def group_body(g, _):
        qsel = q0 + g * L + lane
        qx = plsc.load_gather(cxv, [qsel])
        qy = plsc.load_gather(cyv, [qsel])
        qz = plsc.load_gather(czv, [qsel])

        init = (tuple(jnp.full((L,), BIG, jnp.float32) for _ in range(K))
                + tuple(jnp.full((L,), 2**31 - 1, jnp.int32) for _ in range(K)))

        def blk_body(bb, carry):
            bd = list(carry[:K])
            bi = list(carry[K:])
            base = bb * L
            for r in range(L):
                jv = base + ((lane + r) & (L - 1))
                cx = plsc.load_gather(cxv, [jv])
                cy = plsc.load_gather(cyv, [jv])
                cz = plsc.load_gather(czv, [jv])
                pp = plsc.load_gather(penv, [jv])
                dx = qx - cx
                dy = qy - cy
                dz = qz - cz
                d = ((dx * dx + dy * dy) + dz * dz) + pp
                pred = jnp.any((d < bd[K - 1])
                               | ((d == bd[K - 1]) & (jv < bi[K - 1])))

                def do_insert(op, d=d, jv=jv):
                    nbd, nbi = list(op[0]), list(op[1])
                    cd, ci = d, jv
                    for t in range(K):
                        lt = (cd < nbd[t]) | ((cd == nbd[t]) & (ci < nbi[t]))
                        td = jnp.where(lt, cd, nbd[t])
                        ti = jnp.where(lt, ci, nbi[t])
                        cd = jnp.where(lt, nbd[t], cd)
                        ci = jnp.where(lt, nbi[t], ci)
                        nbd[t] = td
                        nbi[t] = ti
                    return tuple(nbd), tuple(nbi)

                def no_insert(op):
                    return op[0], op[1]

                bdt, bit = lax.cond(pred, do_insert, no_insert,
                                    (tuple(bd), tuple(bi)))
                bd, bi = list(bdt), list(bit)
            return tuple(bd) + tuple(bi)

        final = lax.fori_loop(0, NBLK, blk_body, init, unroll=False)
        bd = final[:K]
        bi = final[K:]

        f = [plsc.load_gather(frv, [jnp.full((L,), k9, jnp.int32),
                                    g * L + lane]) for k9 in range(9)]
        for r in range(K):
            nb = bi[r]
            p = g * (L * K) + lane * K + r
            plsc.store_scatter(idxbuf, [p >> 7, p & 127], nb + batch * N)
            gx = plsc.load_gather(cxv, [nb])
            gy = plsc.load_gather(cyv, [nb])
            gz = plsc.load_gather(czv, [nb])
            ddx = gx - qx
            ddy = gy - qy
            ddz = gz - qz
            cbase = g * (L * K * 3) + lane * (K * 3) + r * 3
            for m in range(3):
                cm = ddx * f[3 * m] + ddy * f[3 * m + 1] + ddz * f[3 * m + 2]
                plsc.store_scatter(coordbuf, [cbase + m], cm)
        return 0

    lax.fori_loop(0, NGRP, group_body, 0, unroll=False)

    pltpu.sync_copy(coordbuf, out_c_hbm.at[pl.ds(wid * (QPT * K * 3),
                                                 QPT * K * 3)])

    bufs = (rows0, rows1)
    sems = (sem0, sem1)
    descs = [None] * IDX_ROWS
    descs[0] = pltpu.async_copy(attr_hbm.at[idxbuf.at[0]], bufs[0], sems[0])
    for j in range(IDX_ROWS):
        if j + 1 < IDX_ROWS:
            descs[j + 1] = pltpu.async_copy(attr_hbm.at[idxbuf.at[j + 1]],
                                            bufs[(j + 1) % 2],
                                            sems[(j + 1) % 2])
        descs[j].wait()
        pltpu.sync_copy(bufs[j % 2],
                        out_a_hbm.at[pl.ds(wid * (QPT * K) + j * 128, 128)])


@jax.jit
def _run(cen, fr9, attr_flat):
    mesh = plsc.VectorSubcoreMesh(core_axis_name="c", subcore_axis_name="s",
                                  num_cores=NC, num_subcores=NS)
    return pl.kernel(
        _body,
        out_type=[
            jax.ShapeDtypeStruct((B * N * K * 3,), jnp.float32),
            jax.ShapeDtypeStruct((B * N * K, D), jnp.float32),
        ],
        mesh=mesh,
        compiler_params=pltpu.CompilerParams(needs_layout_passes=False),
        scratch_types=[
            pltpu.VMEM((N,), jnp.float32),
            pltpu.VMEM((N,), jnp.float32),
            pltpu.VMEM((N,), jnp.float32),
            pltpu.VMEM((N,), jnp.float32),
            pltpu.VMEM((9, QPT), jnp.float32),
            pltpu.VMEM((IDX_ROWS, 128), jnp.int32),
            pltpu.VMEM((QPT * K * 3,), jnp.float32),
            pltpu.VMEM((128, D), jnp.float32),
            pltpu.VMEM((128, D), jnp.float32),
            pltpu.SemaphoreType.DMA,
            pltpu.SemaphoreType.DMA,
        ],
    )(cen, fr9, attr_flat)


def kernel(frame, attributes, mask):
    centers = frame[:, :, 0, :]
    pen = 2000.0 * (1.0 - mask[0][:, :, 1])
    cen = jnp.concatenate(
        [jnp.moveaxis(centers, -1, 1), pen[:, None, :]], axis=1)
    fr9 = jnp.moveaxis(frame[:, :, 1:4, :].reshape(B, N, 9), -1, 1)
    attr_flat = attributes.reshape(B * N, D)
    coords, attrs = _run(cen, fr9, attr_flat)
    return (coords.reshape(B, N, K, 3), attrs.reshape(B, N, K, D))


# speedup vs baseline: 8.9497x; 1.9735x over previous
import functools

import jax
import jax.numpy as jnp
from jax import lax
from jax.experimental import pallas as pl
from jax.experimental.pallas import tpu as pltpu
from jax.experimental.pallas import tpu_sc as plsc

B, N, D, K = 4, 2048, 128, 16
L = 16
NC, NS = 2, 16
NW = NC * NS
QPT = (B * N) // NW
TPB = N // QPT
NBLK = N // L
NGRP = QPT // L
IDX_ROWS = (QPT * K) // 128
BIG = 1e30
IMAX = 2**31 - 1
NSTRIPE = 16
BPS = NBLK // NSTRIPE
CAP = 320


def _body(cen_hbm, fr9_hbm, attr_hbm, out_c_hbm, out_a_hbm,
          cxv, cyv, czv, penv, frv, idxbuf, coordbuf, dbuf, dbucket, jbucket,
          rows0, rows1, sem0, sem1):
    wid = lax.axis_index("s") * NC + lax.axis_index("c")
    batch = wid // TPB
    q0 = (wid % TPB) * QPT

    pltpu.sync_copy(cen_hbm.at[batch, 0], cxv)
    pltpu.sync_copy(cen_hbm.at[batch, 1], cyv)
    pltpu.sync_copy(cen_hbm.at[batch, 2], czv)
    pltpu.sync_copy(cen_hbm.at[batch, 3], penv)
    pltpu.sync_copy(fr9_hbm.at[batch, :, pl.ds(q0, QPT)], frv)

    lane = lax.iota(jnp.int32, L)

    def group_body(g, _):
        qsel = q0 + g * L + lane
        qx = plsc.load_gather(cxv, [qsel])
        qy = plsc.load_gather(cyv, [qsel])
        qz = plsc.load_gather(czv, [qsel])

        def stripe_body(s, u):
            def b8_body(b8, smin):
                bb = s * BPS + b8
                base = bb * L
                for r in range(L):
                    jv = base + ((lane + r) & (L - 1))
                    cx = plsc.load_gather(cxv, [jv])
                    cy = plsc.load_gather(cyv, [jv])
                    cz = plsc.load_gather(czv, [jv])
                    pp = plsc.load_gather(penv, [jv])
                    dx = qx - cx
                    dy = qy - cy
                    dz = qz - cz
                    d = ((dx * dx + dy * dy) + dz * dz) + pp
                    plsc.store_scatter(dbuf, [(bb * L + r) * L + lane], d)
                    smin = jnp.minimum(smin, d)
                return smin
            smin = lax.fori_loop(0, BPS, b8_body,
                                 jnp.full((L,), BIG, jnp.float32))
            return jnp.maximum(u, smin)

        u = lax.fori_loop(0, NSTRIPE, stripe_body,
                          jnp.full((L,), -BIG, jnp.float32))

        def p2_body(bb, cnt):
            base = bb * L
            for r in range(L):
                d = plsc.load_gather(dbuf, [(bb * L + r) * L + lane])
                jv = base + ((lane + r) & (L - 1))
                keep = d <= u
                pos = lane * CAP + jnp.minimum(cnt, CAP - 1)
                plsc.store_scatter(dbucket, [pos], d, mask=keep)
                plsc.store_scatter(jbucket, [pos], jv, mask=keep)
                cnt = cnt + keep.astype(jnp.int32)
            return cnt

        cnt = lax.fori_loop(0, NBLK, p2_body, jnp.zeros((L,), jnp.int32))
        maxcnt = jnp.minimum(jnp.max(cnt), CAP)

        init = (tuple(jnp.full((L,), BIG, jnp.float32) for _ in range(K))
                + tuple(jnp.full((L,), IMAX, jnp.int32) for _ in range(K)))

        def pb_body(t, carry):
            bd = list(carry[:K])
            bi = list(carry[K:])
            valid = t < cnt
            cd = plsc.load_gather(dbucket, [lane * CAP + t])
            ci = plsc.load_gather(jbucket, [lane * CAP + t])
            cd = jnp.where(valid, cd, BIG)
            ci = jnp.where(valid, ci, IMAX)
            for t2 in range(K):
                lt = (cd < bd[t2]) | ((cd == bd[t2]) & (ci < bi[t2]))
                td = jnp.where(lt, cd, bd[t2])
                ti = jnp.where(lt, ci, bi[t2])
                cd = jnp.where(lt, bd[t2], cd)
                ci = jnp.where(lt, bi[t2], ci)
                bd[t2] = td
                bi[t2] = ti
            return tuple(bd) + tuple(bi)

        final = lax.fori_loop(0, maxcnt, pb_body, init, unroll=False)
        bd = final[:K]
        bi = final[K:]

        f = [plsc.load_gather(frv, [jnp.full((L,), k9, jnp.int32),
                                    g * L + lane]) for k9 in range(9)]
        for r in range(K):
            nb = bi[r]
            p = g * (L * K) + lane * K + r
            plsc.store_scatter(idxbuf, [p >> 7, p & 127], nb + batch * N)
            gx = plsc.load_gather(cxv, [nb])
            gy = plsc.load_gather(cyv, [nb])
            gz = plsc.load_gather(czv, [nb])
            ddx = gx - qx
            ddy = gy - qy
            ddz = gz - qz
            cbase = g * (L * K * 3) + lane * (K * 3) + r * 3
            for m in range(3):
                cm = ddx * f[3 * m] + ddy * f[3 * m + 1] + ddz * f[3 * m + 2]
                plsc.store_scatter(coordbuf, [cbase + m], cm)
        return 0

    lax.fori_loop(0, NGRP, group_body, 0, unroll=False)

    pltpu.sync_copy(coordbuf, out_c_hbm.at[pl.ds(wid * (QPT * K * 3),
                                                 QPT * K * 3)])

    bufs = (rows0, rows1)
    sems = (sem0, sem1)
    descs = [None] * IDX_ROWS
    descs[0] = pltpu.async_copy(attr_hbm.at[idxbuf.at[0]], bufs[0], sems[0])
    for j in range(IDX_ROWS):
        if j + 1 < IDX_ROWS:
            descs[j + 1] = pltpu.async_copy(attr_hbm.at[idxbuf.at[j + 1]],
                                            bufs[(j + 1) % 2],
                                            sems[(j + 1) % 2])
        descs[j].wait()
        pltpu.sync_copy(bufs[j % 2],
                        out_a_hbm.at[pl.ds(wid * (QPT * K) + j * 128, 128)])


@jax.jit
def _run(cen, fr9, attr_flat):
    mesh = plsc.VectorSubcoreMesh(core_axis_name="c", subcore_axis_name="s",
                                  num_cores=NC, num_subcores=NS)
    return pl.kernel(
        _body,
        out_type=[
            jax.ShapeDtypeStruct((B * N * K * 3,), jnp.float32),
            jax.ShapeDtypeStruct((B * N * K, D), jnp.float32),
        ],
        mesh=mesh,
        compiler_params=pltpu.CompilerParams(needs_layout_passes=False),
        scratch_types=[
            pltpu.VMEM((N,), jnp.float32),
            pltpu.VMEM((N,), jnp.float32),
            pltpu.VMEM((N,), jnp.float32),
            pltpu.VMEM((N,), jnp.float32),
            pltpu.VMEM((9, QPT), jnp.float32),
            pltpu.VMEM((IDX_ROWS, 128), jnp.int32),
            pltpu.VMEM((QPT * K * 3,), jnp.float32),
            pltpu.VMEM((N * L,), jnp.float32),
            pltpu.VMEM((L * CAP,), jnp.float32),
            pltpu.VMEM((L * CAP,), jnp.int32),
            pltpu.VMEM((128, D), jnp.float32),
            pltpu.VMEM((128, D), jnp.float32),
            pltpu.SemaphoreType.DMA,
            pltpu.SemaphoreType.DMA,
        ],
    )(cen, fr9, attr_flat)


def kernel(frame, attributes, mask):
    centers = frame[:, :, 0, :]
    pen = 2000.0 * (1.0 - mask[0][:, :, 1])
    cen = jnp.concatenate(
        [jnp.moveaxis(centers, -1, 1), pen[:, None, :]], axis=1)
    fr9 = jnp.moveaxis(frame[:, :, 1:4, :].reshape(B, N, 9), -1, 1)
    attr_flat = attributes.reshape(B * N, D)
    coords, attrs = _run(cen, fr9, attr_flat)
    return (coords.reshape(B, N, K, 3), attrs.reshape(B, N, K, D))


# speedup vs baseline: 12.0323x; 1.3444x over previous
import functools

import jax
import jax.numpy as jnp
from jax import lax
from jax.experimental import pallas as pl
from jax.experimental.pallas import tpu as pltpu
from jax.experimental.pallas import tpu_sc as plsc

B, N, D, K = 4, 2048, 128, 16
L = 16
NC, NS = 2, 16
NW = NC * NS
QPT = (B * N) // NW
TPB = N // QPT
NBLK = N // L
NGRP = QPT // L
IDX_ROWS = (QPT * K) // 128
BIG = 1e30
IMAX = 2**31 - 1
NSTRIPE = 16
BPS = NBLK // NSTRIPE
CAP = 320


def _body(cen_hbm, fr9_hbm, attr_hbm, out_c_hbm, out_a_hbm,
          cxv, cyv, czv, penv, frv, idxbuf, coordbuf, dbuf, dbucket, jbucket,
          rows0, rows1, sem0, sem1):
    wid = lax.axis_index("s") * NC + lax.axis_index("c")
    batch = wid // TPB
    q0 = (wid % TPB) * QPT

    pltpu.sync_copy(cen_hbm.at[batch, 0], cxv)
    pltpu.sync_copy(cen_hbm.at[batch, 1], cyv)
    pltpu.sync_copy(cen_hbm.at[batch, 2], czv)
    pltpu.sync_copy(cen_hbm.at[batch, 3], penv)
    pltpu.sync_copy(fr9_hbm.at[batch, :, pl.ds(q0, QPT)], frv)

    lane = lax.iota(jnp.int32, L)

    def group_body(g, _):
        qsel = q0 + g * L + lane
        qx = plsc.load_gather(cxv, [qsel])
        qy = plsc.load_gather(cyv, [qsel])
        qz = plsc.load_gather(czv, [qsel])

        def stripe_body(s, u):
            @plsc.parallel_loop(0, BPS, carry=jnp.full((L,), BIG, jnp.float32))
            def smin(b8, smin_c):
                bb = s * BPS + b8
                base = bb * L
                for r in range(L):
                    jv = base + ((lane + r) & (L - 1))
                    cx = plsc.load_gather(cxv, [jv])
                    cy = plsc.load_gather(cyv, [jv])
                    cz = plsc.load_gather(czv, [jv])
                    pp = plsc.load_gather(penv, [jv])
                    dx = qx - cx
                    dy = qy - cy
                    dz = qz - cz
                    d = ((dx * dx + dy * dy) + dz * dz) + pp
                    plsc.store_scatter(dbuf, [(bb * L + r) * L + lane], d)
                    smin_c = jnp.minimum(smin_c, d)
                return smin_c
            return jnp.maximum(u, smin)

        u = lax.fori_loop(0, NSTRIPE, stripe_body,
                          jnp.full((L,), -BIG, jnp.float32))

        @plsc.parallel_loop(0, NBLK, carry=jnp.zeros((L,), jnp.int32))
        def cnt(bb, cnt_c):
            base = bb * L
            for r in range(L):
                d = plsc.load_gather(dbuf, [(bb * L + r) * L + lane])
                jv = base + ((lane + r) & (L - 1))
                keep = d <= u
                pos = lane * CAP + jnp.minimum(cnt_c, CAP - 1)
                plsc.store_scatter(dbucket, [pos], d, mask=keep)
                plsc.store_scatter(jbucket, [pos], jv, mask=keep)
                cnt_c = cnt_c + keep.astype(jnp.int32)
            return cnt_c
        maxcnt = jnp.minimum(jnp.max(cnt), CAP)

        init = (tuple(jnp.full((L,), BIG, jnp.float32) for _ in range(K))
                + tuple(jnp.full((L,), IMAX, jnp.int32) for _ in range(K)))

        def pb_body(tb, carry):
            bd = list(carry[:K])
            bi = list(carry[K:])
            cands = []
            for uu in range(4):
                t = tb * 4 + uu
                valid = t < cnt
                cd = plsc.load_gather(dbucket, [lane * CAP + t])
                ci = plsc.load_gather(jbucket, [lane * CAP + t])
                cands.append((jnp.where(valid, cd, BIG),
                              jnp.where(valid, ci, IMAX)))
            for cd, ci in cands:
                for t2 in range(K):
                    lt = (cd < bd[t2]) | ((cd == bd[t2]) & (ci < bi[t2]))
                    td = jnp.where(lt, cd, bd[t2])
                    ti = jnp.where(lt, ci, bi[t2])
                    cd = jnp.where(lt, bd[t2], cd)
                    ci = jnp.where(lt, bi[t2], ci)
                    bd[t2] = td
                    bi[t2] = ti
            return tuple(bd) + tuple(bi)

        final = lax.fori_loop(0, (maxcnt + 3) // 4, pb_body, init,
                              unroll=False)
        bd = final[:K]
        bi = final[K:]

        f = [plsc.load_gather(frv, [jnp.full((L,), k9, jnp.int32),
                                    g * L + lane]) for k9 in range(9)]
        for r in range(K):
            nb = bi[r]
            p = g * (L * K) + lane * K + r
            plsc.store_scatter(idxbuf, [p >> 7, p & 127], nb + batch * N)
            gx = plsc.load_gather(cxv, [nb])
            gy = plsc.load_gather(cyv, [nb])
            gz = plsc.load_gather(czv, [nb])
            ddx = gx - qx
            ddy = gy - qy
            ddz = gz - qz
            cbase = g * (L * K * 3) + lane * (K * 3) + r * 3
            for m in range(3):
                cm = ddx * f[3 * m] + ddy * f[3 * m + 1] + ddz * f[3 * m + 2]
                plsc.store_scatter(coordbuf, [cbase + m], cm)
        return 0

    lax.fori_loop(0, NGRP, group_body, 0, unroll=False)

    pltpu.sync_copy(coordbuf, out_c_hbm.at[pl.ds(wid * (QPT * K * 3),
                                                 QPT * K * 3)])

    bufs = (rows0, rows1)
    sems = (sem0, sem1)
    descs = [None] * IDX_ROWS
    descs[0] = pltpu.async_copy(attr_hbm.at[idxbuf.at[0]], bufs[0], sems[0])
    for j in range(IDX_ROWS):
        if j + 1 < IDX_ROWS:
            descs[j + 1] = pltpu.async_copy(attr_hbm.at[idxbuf.at[j + 1]],
                                            bufs[(j + 1) % 2],
                                            sems[(j + 1) % 2])
        descs[j].wait()
        pltpu.sync_copy(bufs[j % 2],
                        out_a_hbm.at[pl.ds(wid * (QPT * K) + j * 128, 128)])


@jax.jit
def _run(cen, fr9, attr_flat):
    mesh = plsc.VectorSubcoreMesh(core_axis_name="c", subcore_axis_name="s",
                                  num_cores=NC, num_subcores=NS)
    return pl.kernel(
        _body,
        out_type=[
            jax.ShapeDtypeStruct((B * N * K * 3,), jnp.float32),
            jax.ShapeDtypeStruct((B * N * K, D), jnp.float32),
        ],
        mesh=mesh,
        compiler_params=pltpu.CompilerParams(needs_layout_passes=False),
        scratch_types=[
            pltpu.VMEM((N,), jnp.float32),
            pltpu.VMEM((N,), jnp.float32),
            pltpu.VMEM((N,), jnp.float32),
            pltpu.VMEM((N,), jnp.float32),
            pltpu.VMEM((9, QPT), jnp.float32),
            pltpu.VMEM((IDX_ROWS, 128), jnp.int32),
            pltpu.VMEM((QPT * K * 3,), jnp.float32),
            pltpu.VMEM((N * L,), jnp.float32),
            pltpu.VMEM((L * CAP,), jnp.float32),
            pltpu.VMEM((L * CAP,), jnp.int32),
            pltpu.VMEM((128, D), jnp.float32),
            pltpu.VMEM((128, D), jnp.float32),
            pltpu.SemaphoreType.DMA,
            pltpu.SemaphoreType.DMA,
        ],
    )(cen, fr9, attr_flat)


def kernel(frame, attributes, mask):
    centers = frame[:, :, 0, :]
    pen = 2000.0 * (1.0 - mask[0][:, :, 1])
    cen = jnp.concatenate(
        [jnp.moveaxis(centers, -1, 1), pen[:, None, :]], axis=1)
    fr9 = jnp.moveaxis(frame[:, :, 1:4, :].reshape(B, N, 9), -1, 1)
    attr_flat = attributes.reshape(B * N, D)
    coords, attrs = _run(cen, fr9, attr_flat)
    return (coords.reshape(B, N, K, 3), attrs.reshape(B, N, K, D))


# speedup vs baseline: 12.7031x; 1.0557x over previous
import functools

import jax
import jax.numpy as jnp
from jax import lax
from jax.experimental import pallas as pl
from jax.experimental.pallas import tpu as pltpu
from jax.experimental.pallas import tpu_sc as plsc

B, N, D, K = 4, 2048, 128, 16
L = 16
NC, NS = 2, 16
NW = NC * NS
QPT = (B * N) // NW
TPB = N // QPT
NBLK = N // L
NGRP = QPT // L
IDX_ROWS = (QPT * K) // 128
BIG = 1e30
IMAX = 2**31 - 1
NSTRIPE = 16
BPS = NBLK // NSTRIPE
CAP = 320


def _body(cen_hbm, fr9_hbm, attr_hbm, out_c_hbm, out_a_hbm,
          cxv, cyv, czv, penv, frv, idxbuf, coordbuf, dbuf, jbucket,
          rows0, rows1, sem0, sem1):
    wid = lax.axis_index("s") * NC + lax.axis_index("c")
    batch = wid // TPB
    q0 = (wid % TPB) * QPT

    pltpu.sync_copy(cen_hbm.at[batch, 0], cxv)
    pltpu.sync_copy(cen_hbm.at[batch, 1], cyv)
    pltpu.sync_copy(cen_hbm.at[batch, 2], czv)
    pltpu.sync_copy(cen_hbm.at[batch, 3], penv)
    pltpu.sync_copy(fr9_hbm.at[batch, :, pl.ds(q0, QPT)], frv)

    lane = lax.iota(jnp.int32, L)

    def group_body(g, _):
        qsel = q0 + g * L + lane
        qx = plsc.load_gather(cxv, [qsel])
        qy = plsc.load_gather(cyv, [qsel])
        qz = plsc.load_gather(czv, [qsel])

        def stripe_body(s, u):
            @plsc.parallel_loop(0, BPS, carry=jnp.full((L,), BIG, jnp.float32))
            def smin(b8, smin_c):
                bb = s * BPS + b8
                base = bb * L
                for r in range(L):
                    jv = base + ((lane + r) & (L - 1))
                    cx = plsc.load_gather(cxv, [jv])
                    cy = plsc.load_gather(cyv, [jv])
                    cz = plsc.load_gather(czv, [jv])
                    pp = plsc.load_gather(penv, [jv])
                    dx = qx - cx
                    dy = qy - cy
                    dz = qz - cz
                    d = ((dx * dx + dy * dy) + dz * dz) + pp
                    plsc.store_scatter(dbuf, [(bb * L + r) * L + lane], d)
                    smin_c = jnp.minimum(smin_c, d)
                return smin_c
            return jnp.maximum(u, smin)

        u = lax.fori_loop(0, NSTRIPE, stripe_body,
                          jnp.full((L,), -BIG, jnp.float32))

        @plsc.parallel_loop(0, NBLK, carry=jnp.zeros((L,), jnp.int32))
        def cnt(bb, cnt_c):
            for r in range(L):
                addr = (bb * L + r) * L + lane
                d = plsc.load_gather(dbuf, [addr])
                keep = d <= u
                pos = lane * CAP + jnp.minimum(cnt_c, CAP - 1)
                plsc.store_scatter(jbucket, [pos], addr, mask=keep)
                cnt_c = cnt_c + keep.astype(jnp.int32)
            return cnt_c
        maxcnt = jnp.minimum(jnp.max(cnt), CAP)

        init = (tuple(jnp.full((L,), BIG, jnp.float32) for _ in range(K))
                + tuple(jnp.full((L,), IMAX, jnp.int32) for _ in range(K)))

        def pb_body(tb, carry):
            bd = list(carry[:K])
            bi = list(carry[K:])
            cands = []
            for uu in range(4):
                t = tb * 4 + uu
                valid = t < cnt
                addr = plsc.load_gather(jbucket, [lane * CAP + t]) & (N * L - 1)
                cd = plsc.load_gather(dbuf, [addr])
                s = addr >> 4
                ci = ((s >> 4) * L) + ((lane + (s & (L - 1))) & (L - 1))
                cands.append((jnp.where(valid, cd, BIG),
                              jnp.where(valid, ci, IMAX)))
            for cd, ci in cands:
                for t2 in range(K):
                    lt = (cd < bd[t2]) | ((cd == bd[t2]) & (ci < bi[t2]))
                    td = jnp.where(lt, cd, bd[t2])
                    ti = jnp.where(lt, ci, bi[t2])
                    cd = jnp.where(lt, bd[t2], cd)
                    ci = jnp.where(lt, bi[t2], ci)
                    bd[t2] = td
                    bi[t2] = ti
            return tuple(bd) + tuple(bi)

        final = lax.fori_loop(0, (maxcnt + 3) // 4, pb_body, init,
                              unroll=False)
        bd = final[:K]
        bi = final[K:]

        f = [plsc.load_gather(frv, [jnp.full((L,), k9, jnp.int32),
                                    g * L + lane]) for k9 in range(9)]
        for r in range(K):
            nb = bi[r]
            p = g * (L * K) + lane * K + r
            plsc.store_scatter(idxbuf, [p >> 7, p & 127], nb + batch * N)
            gx = plsc.load_gather(cxv, [nb])
            gy = plsc.load_gather(cyv, [nb])
            gz = plsc.load_gather(czv, [nb])
            ddx = gx - qx
            ddy = gy - qy
            ddz = gz - qz
            cbase = g * (L * K * 3) + lane * (K * 3) + r * 3
            for m in range(3):
                cm = ddx * f[3 * m] + ddy * f[3 * m + 1] + ddz * f[3 * m + 2]
                plsc.store_scatter(coordbuf, [cbase + m], cm)
        return 0

    lax.fori_loop(0, NGRP, group_body, 0, unroll=False)

    pltpu.sync_copy(coordbuf, out_c_hbm.at[pl.ds(wid * (QPT * K * 3),
                                                 QPT * K * 3)])

    bufs = (rows0, rows1)
    sems = (sem0, sem1)
    descs = [None] * IDX_ROWS
    descs[0] = pltpu.async_copy(attr_hbm.at[idxbuf.at[0]], bufs[0], sems[0])
    for j in range(IDX_ROWS):
        if j + 1 < IDX_ROWS:
            descs[j + 1] = pltpu.async_copy(attr_hbm.at[idxbuf.at[j + 1]],
                                            bufs[(j + 1) % 2],
                                            sems[(j + 1) % 2])
        descs[j].wait()
        pltpu.sync_copy(bufs[j % 2],
                        out_a_hbm.at[pl.ds(wid * (QPT * K) + j * 128, 128)])


@jax.jit
def _run(cen, fr9, attr_flat):
    mesh = plsc.VectorSubcoreMesh(core_axis_name="c", subcore_axis_name="s",
                                  num_cores=NC, num_subcores=NS)
    return pl.kernel(
        _body,
        out_type=[
            jax.ShapeDtypeStruct((B * N * K * 3,), jnp.float32),
            jax.ShapeDtypeStruct((B * N * K, D), jnp.float32),
        ],
        mesh=mesh,
        compiler_params=pltpu.CompilerParams(needs_layout_passes=False),
        scratch_types=[
            pltpu.VMEM((N,), jnp.float32),
            pltpu.VMEM((N,), jnp.float32),
            pltpu.VMEM((N,), jnp.float32),
            pltpu.VMEM((N,), jnp.float32),
            pltpu.VMEM((9, QPT), jnp.float32),
            pltpu.VMEM((IDX_ROWS, 128), jnp.int32),
            pltpu.VMEM((QPT * K * 3,), jnp.float32),
            pltpu.VMEM((N * L,), jnp.float32),
            pltpu.VMEM((L * CAP,), jnp.int32),
            pltpu.VMEM((128, D), jnp.float32),
            pltpu.VMEM((128, D), jnp.float32),
            pltpu.SemaphoreType.DMA,
            pltpu.SemaphoreType.DMA,
        ],
    )(cen, fr9, attr_flat)


def kernel(frame, attributes, mask):
    centers = frame[:, :, 0, :]
    pen = 2000.0 * (1.0 - mask[0][:, :, 1])
    cen = jnp.concatenate(
        [jnp.moveaxis(centers, -1, 1), pen[:, None, :]], axis=1)
    fr9 = jnp.moveaxis(frame[:, :, 1:4, :].reshape(B, N, 9), -1, 1)
    attr_flat = attributes.reshape(B * N, D)
    coords, attrs = _run(cen, fr9, attr_flat)
    return (coords.reshape(B, N, K, 3), attrs.reshape(B, N, K, D))


# speedup vs baseline: 16.0805x; 1.2659x over previous
import functools

import jax
import jax.numpy as jnp
from jax import lax
from jax.experimental import pallas as pl
from jax.experimental.pallas import tpu as pltpu
from jax.experimental.pallas import tpu_sc as plsc

B, N, D, K = 4, 2048, 128, 16
L = 16
NC, NS = 2, 16
NW = NC * NS
QPT = (B * N) // NW
TPB = N // QPT
NBLK = N // L
NGRP = QPT // L
IDX_ROWS = (QPT * K) // 128
BIG = 1e30
IMAX = 2**31 - 1
NSTRIPE = 16
BPS = NBLK // NSTRIPE
CAP = 320


def _body(cen_hbm, fr9_hbm, attr_hbm, out_c_hbm, out_a_hbm,
          cxv, cyv, czv, penv, frv, idxbuf, coordbuf, dbuf, jbucket,
          rows0, rows1, sem0, sem1):
    wid = lax.axis_index("s") * NC + lax.axis_index("c")
    batch = wid // TPB
    q0 = (wid % TPB) * QPT

    pltpu.sync_copy(cen_hbm.at[batch, 0], cxv)
    pltpu.sync_copy(cen_hbm.at[batch, 1], cyv)
    pltpu.sync_copy(cen_hbm.at[batch, 2], czv)
    pltpu.sync_copy(cen_hbm.at[batch, 3], penv)
    pltpu.sync_copy(fr9_hbm.at[batch, :, pl.ds(q0, QPT)], frv)

    lane = lax.iota(jnp.int32, L)

    def group_body(g, _):
        qsel = q0 + g * L + lane
        qx = plsc.load_gather(cxv, [qsel])
        qy = plsc.load_gather(cyv, [qsel])
        qz = plsc.load_gather(czv, [qsel])

        def stripe_body(s, u):
            @plsc.parallel_loop(0, BPS, carry=jnp.full((L,), BIG, jnp.float32))
            def smin(b8, smin_c):
                bb = s * BPS + b8
                base = bb * L
                for r in range(L):
                    jv = base + ((lane + r) & (L - 1))
                    cx = plsc.load_gather(cxv, [jv])
                    cy = plsc.load_gather(cyv, [jv])
                    cz = plsc.load_gather(czv, [jv])
                    pp = plsc.load_gather(penv, [jv])
                    dx = qx - cx
                    dy = qy - cy
                    dz = qz - cz
                    d = ((dx * dx + dy * dy) + dz * dz) + pp
                    plsc.store_scatter(dbuf, [(bb * L + r) * L + lane], d)
                    smin_c = jnp.minimum(smin_c, d)
                return smin_c
            return jnp.maximum(u, smin)

        u = lax.fori_loop(0, NSTRIPE, stripe_body,
                          jnp.full((L,), -BIG, jnp.float32))

        @plsc.parallel_loop(0, NBLK, carry=jnp.zeros((L,), jnp.int32))
        def cnt(bb, cnt_c):
            ds = [dbuf[pl.ds((bb * L + r) * L, L)] for r in range(L)]
            keeps = [d <= u for d in ds]
            for r in range(L):
                pos = lane * CAP + jnp.minimum(cnt_c, CAP - 1)
                plsc.store_scatter(jbucket, [pos], (bb * L + r) * L + lane,
                                   mask=keeps[r])
                cnt_c = cnt_c + keeps[r].astype(jnp.int32)
            return cnt_c
        maxcnt = jnp.minimum(jnp.max(cnt), CAP)

        init = (tuple(jnp.full((L,), BIG, jnp.float32) for _ in range(K))
                + tuple(jnp.full((L,), IMAX, jnp.int32) for _ in range(K)))

        def pb_body(tb, carry):
            bd = list(carry[:K])
            bi = list(carry[K:])
            cands = []
            for uu in range(4):
                t = tb * 4 + uu
                valid = t < cnt
                addr = plsc.load_gather(jbucket, [lane * CAP + t]) & (N * L - 1)
                cd = plsc.load_gather(dbuf, [addr])
                s = addr >> 4
                ci = ((s >> 4) * L) + ((lane + (s & (L - 1))) & (L - 1))
                cands.append((jnp.where(valid, cd, BIG),
                              jnp.where(valid, ci, IMAX)))
            for cd, ci in cands:
                for t2 in range(K):
                    lt = (cd < bd[t2]) | ((cd == bd[t2]) & (ci < bi[t2]))
                    td = jnp.where(lt, cd, bd[t2])
                    ti = jnp.where(lt, ci, bi[t2])
                    cd = jnp.where(lt, bd[t2], cd)
                    ci = jnp.where(lt, bi[t2], ci)
                    bd[t2] = td
                    bi[t2] = ti
            return tuple(bd) + tuple(bi)

        final = lax.fori_loop(0, (maxcnt + 3) // 4, pb_body, init,
                              unroll=False)
        bd = final[:K]
        bi = final[K:]

        f = [plsc.load_gather(frv, [jnp.full((L,), k9, jnp.int32),
                                    g * L + lane]) for k9 in range(9)]
        for r in range(K):
            nb = bi[r]
            p = g * (L * K) + lane * K + r
            plsc.store_scatter(idxbuf, [p >> 7, p & 127], nb + batch * N)
            gx = plsc.load_gather(cxv, [nb])
            gy = plsc.load_gather(cyv, [nb])
            gz = plsc.load_gather(czv, [nb])
            ddx = gx - qx
            ddy = gy - qy
            ddz = gz - qz
            cbase = g * (L * K * 3) + lane * (K * 3) + r * 3
            for m in range(3):
                cm = ddx * f[3 * m] + ddy * f[3 * m + 1] + ddz * f[3 * m + 2]
                plsc.store_scatter(coordbuf, [cbase + m], cm)
        return 0

    lax.fori_loop(0, NGRP, group_body, 0, unroll=False)

    pltpu.sync_copy(coordbuf, out_c_hbm.at[pl.ds(wid * (QPT * K * 3),
                                                 QPT * K * 3)])

    bufs = (rows0, rows1)
    sems = (sem0, sem1)
    descs = [None] * IDX_ROWS
    descs[0] = pltpu.async_copy(attr_hbm.at[idxbuf.at[0]], bufs[0], sems[0])
    for j in range(IDX_ROWS):
        if j + 1 < IDX_ROWS:
            descs[j + 1] = pltpu.async_copy(attr_hbm.at[idxbuf.at[j + 1]],
                                            bufs[(j + 1) % 2],
                                            sems[(j + 1) % 2])
        descs[j].wait()
        pltpu.sync_copy(bufs[j % 2],
                        out_a_hbm.at[pl.ds(wid * (QPT * K) + j * 128, 128)])


@jax.jit
def _run(cen, fr9, attr_flat):
    mesh = plsc.VectorSubcoreMesh(core_axis_name="c", subcore_axis_name="s",
                                  num_cores=NC, num_subcores=NS)
    return pl.kernel(
        _body,
        out_type=[
            jax.ShapeDtypeStruct((B * N * K * 3,), jnp.float32),
            jax.ShapeDtypeStruct((B * N * K, D), jnp.float32),
        ],
        mesh=mesh,
        compiler_params=pltpu.CompilerParams(needs_layout_passes=False),
        scratch_types=[
            pltpu.VMEM((N,), jnp.float32),
            pltpu.VMEM((N,), jnp.float32),
            pltpu.VMEM((N,), jnp.float32),
            pltpu.VMEM((N,), jnp.float32),
            pltpu.VMEM((9, QPT), jnp.float32),
            pltpu.VMEM((IDX_ROWS, 128), jnp.int32),
            pltpu.VMEM((QPT * K * 3,), jnp.float32),
            pltpu.VMEM((N * L,), jnp.float32),
            pltpu.VMEM((L * CAP,), jnp.int32),
            pltpu.VMEM((128, D), jnp.float32),
            pltpu.VMEM((128, D), jnp.float32),
            pltpu.SemaphoreType.DMA,
            pltpu.SemaphoreType.DMA,
        ],
    )(cen, fr9, attr_flat)


def kernel(frame, attributes, mask):
    centers = frame[:, :, 0, :]
    pen = 2000.0 * (1.0 - mask[0][:, :, 1])
    cen = jnp.concatenate(
        [jnp.moveaxis(centers, -1, 1), pen[:, None, :]], axis=1)
    fr9 = jnp.moveaxis(frame[:, :, 1:4, :].reshape(B, N, 9), -1, 1)
    attr_flat = attributes.reshape(B * N, D)
    coords, attrs = _run(cen, fr9, attr_flat)
    return (coords.reshape(B, N, K, 3), attrs.reshape(B, N, K, D))
